# Initial kernel scaffold; baseline (speedup 1.0000x reference)
import jax
import jax.numpy as jnp
from jax.experimental import pallas as pl


def kernel(x, edge_index, edge_weight, embedding, W, b):
    raise NotImplementedError("write your pallas kernel here")


# speedup vs baseline: 3.5374x; 3.5374x over previous
import functools

import jax
import jax.numpy as jnp
from jax import lax
from jax.experimental import pallas as pl
from jax.experimental.pallas import tpu as pltpu
from jax.experimental.pallas import tpu_sc as plsc

LAMDA = 0.8
ALPHA = (1.0 - LAMDA) / LAMDA

NC = 2
NS = 16
GW = 128


def _matmul_body(emb_ref, w_ref, b_ref, h_ref, h4_ref):
    h = jax.lax.dot_general(
        emb_ref[...], w_ref[...], (((1,), (0,)), ((), ())),
        precision=jax.lax.Precision.HIGHEST,
        preferred_element_type=jnp.float32) + b_ref[...]
    h_ref[...] = h
    h4_ref[...] = ALPHA * h


def _normalize_body(r_ref, o_ref):
    r = r_ref[...]
    norm = jnp.sqrt(jnp.sum(r * r, axis=1, keepdims=True))
    o_ref[...] = r / jnp.maximum(norm, 1e-12)


def _propagate_body(n_nodes, n_groups, b_groups, dh,
                    h_hbm, h4_hbm, src_hbm, dst_hbm, w_hbm, x_hbm, out_hbm,
                    hs, acc, idx_s, idx_d, wbuf, rows, xq, obuf):
    cid = lax.axis_index("c")
    sid = lax.axis_index("s")
    col0 = cid * dh
    rows_per = n_nodes // NS

    r0 = sid * rows_per
    pltpu.sync_copy(h_hbm.at[pl.ds(r0, rows_per), pl.ds(col0, dh)],
                    hs.at[pl.ds(r0, rows_per)])
    pltpu.sync_copy(h4_hbm.at[pl.ds(r0, rows_per), pl.ds(col0, dh)],
                    acc.at[pl.ds(r0, rows_per)])
    plsc.subcore_barrier()

    @pl.loop(sid, n_groups, step=NS)
    def _(g):
        pltpu.sync_copy(src_hbm.at[pl.ds(g, 1)], idx_s)
        pltpu.sync_copy(dst_hbm.at[pl.ds(g, 1)], idx_d)
        pltpu.sync_copy(w_hbm.at[pl.ds(g, 1)], wbuf)
        pltpu.sync_copy(hs.at[idx_s.at[0]], rows)

        @pl.loop(0, GW)
        def _(e):
            z16 = jnp.zeros((16,), jnp.int32)
            wv = plsc.load_gather(wbuf, [z16, jnp.broadcast_to(e, (16,))])
            for j in range(dh // 16):
                sl = pl.ds(j * 16, 16)
                rows[e, sl] = rows[e, sl] * wv

        pltpu.sync_copy(rows, acc.at[idx_d.at[0]], add=True)

    plsc.subcore_barrier()

    @pl.loop(sid, b_groups, step=NS)
    def _(g):
        pltpu.sync_copy(x_hbm.at[pl.ds(g, 1)], xq)
        pltpu.sync_copy(acc.at[xq.at[0]], obuf)
        pltpu.sync_copy(obuf, out_hbm.at[pl.ds(g * GW, GW), pl.ds(col0, dh)])


def kernel(x, edge_index, edge_weight, embedding, W, b):
    n_nodes, d_in = embedding.shape
    d_out = W.shape[1]
    n_edges = edge_weight.shape[0]
    bsz = x.shape[0]
    dh = d_out // NC

    h, h4 = pl.pallas_call(
        _matmul_body,
        grid=(10,),
        in_specs=[
            pl.BlockSpec((n_nodes // 10, d_in), lambda i: (i, 0)),
            pl.BlockSpec((d_in, d_out), lambda i: (0, 0)),
            pl.BlockSpec((1, d_out), lambda i: (0, 0)),
        ],
        out_specs=[
            pl.BlockSpec((n_nodes // 10, d_out), lambda i: (i, 0)),
            pl.BlockSpec((n_nodes // 10, d_out), lambda i: (i, 0)),
        ],
        out_shape=[
            jax.ShapeDtypeStruct((n_nodes, d_out), jnp.float32),
            jax.ShapeDtypeStruct((n_nodes, d_out), jnp.float32),
        ],
    )(embedding, W, b.reshape(1, d_out))

    n_groups = n_edges // GW
    b_groups = bsz // GW
    src2 = edge_index[0].reshape(n_groups, GW)
    dst2 = edge_index[1].reshape(n_groups, GW)
    w2 = edge_weight.reshape(n_groups, GW)
    x2 = x.reshape(b_groups, GW)

    mesh = plsc.VectorSubcoreMesh(core_axis_name="c", subcore_axis_name="s")
    propagate = pl.kernel(
        functools.partial(_propagate_body, n_nodes, n_groups, b_groups, dh),
        out_type=jax.ShapeDtypeStruct((bsz, d_out), jnp.float32),
        mesh=mesh,
        compiler_params=pltpu.CompilerParams(
            use_tc_tiling_on_sc=False, needs_layout_passes=False),
        scratch_types=[
            pltpu.VMEM_SHARED((n_nodes, dh), jnp.float32),
            pltpu.VMEM_SHARED((n_nodes, dh), jnp.float32),
            pltpu.VMEM((1, GW), jnp.int32),
            pltpu.VMEM((1, GW), jnp.int32),
            pltpu.VMEM((1, GW), jnp.float32),
            pltpu.VMEM((GW, dh), jnp.float32),
            pltpu.VMEM((1, GW), jnp.int32),
            pltpu.VMEM((GW, dh), jnp.float32),
        ],
    )
    rows = propagate(h, h4, src2, dst2, w2, x2)

    out = pl.pallas_call(
        _normalize_body,
        grid=(16,),
        in_specs=[pl.BlockSpec((bsz // 16, d_out), lambda i: (i, 0))],
        out_specs=pl.BlockSpec((bsz // 16, d_out), lambda i: (i, 0)),
        out_shape=jax.ShapeDtypeStruct((bsz, d_out), jnp.float32),
    )(rows)
    return out


# speedup vs baseline: 3.6539x; 1.0329x over previous
import functools

import jax
import jax.numpy as jnp
from jax import lax
from jax.experimental import pallas as pl
from jax.experimental.pallas import tpu as pltpu
from jax.experimental.pallas import tpu_sc as plsc

LAMDA = 0.8
ALPHA = (1.0 - LAMDA) / LAMDA

NC = 2
NS = 16
GW = 128


def _matmul_body(emb_ref, w_ref, b_ref, h_ref, h4_ref):
    h = jax.lax.dot_general(
        emb_ref[...], w_ref[...], (((1,), (0,)), ((), ())),
        precision=jax.lax.Precision.HIGHEST,
        preferred_element_type=jnp.float32) + b_ref[...]
    h_ref[...] = h
    h4_ref[...] = ALPHA * h


def _normalize_body(r_ref, o_ref):
    r = r_ref[...]
    norm = jnp.sqrt(jnp.sum(r * r, axis=1, keepdims=True))
    o_ref[...] = r / jnp.maximum(norm, 1e-12)


def _propagate_body(n_nodes, n_groups, b_groups, dh,
                    h_hbm, h4_hbm, src_hbm, dst_hbm, w_hbm, x_hbm, out_hbm,
                    hs, acc, idx_s, idx_d, wbuf, rows, xq, obuf):
    cid = lax.axis_index("c")
    sid = lax.axis_index("s")
    col0 = cid * dh
    rows_per = n_nodes // NS

    r0 = sid * rows_per
    pltpu.sync_copy(h_hbm.at[pl.ds(r0, rows_per), pl.ds(col0, dh)],
                    hs.at[pl.ds(r0, rows_per)])
    pltpu.sync_copy(h4_hbm.at[pl.ds(r0, rows_per), pl.ds(col0, dh)],
                    acc.at[pl.ds(r0, rows_per)])
    plsc.subcore_barrier()

    @pl.loop(sid, n_groups, step=NS)
    def _(g):
        pltpu.sync_copy(src_hbm.at[pl.ds(g, 1)], idx_s)
        pltpu.sync_copy(dst_hbm.at[pl.ds(g, 1)], idx_d)
        pltpu.sync_copy(w_hbm.at[pl.ds(g, 1)], wbuf)
        pltpu.sync_copy(hs.at[idx_s.at[0]], rows)

        @pl.loop(0, GW, step=4)
        def _(e0):
            z16 = jnp.zeros((16,), jnp.int32)
            for u in range(4):
                e = e0 + u
                wv = plsc.load_gather(wbuf, [z16, jnp.broadcast_to(e, (16,))])
                for j in range(dh // 16):
                    sl = pl.ds(j * 16, 16)
                    rows[e, sl] = rows[e, sl] * wv

        pltpu.sync_copy(rows, acc.at[idx_d.at[0]], add=True)

    plsc.subcore_barrier()

    @pl.loop(sid, b_groups, step=NS)
    def _(g):
        pltpu.sync_copy(x_hbm.at[pl.ds(g, 1)], xq)
        pltpu.sync_copy(acc.at[xq.at[0]], obuf)
        pltpu.sync_copy(obuf, out_hbm.at[pl.ds(g * GW, GW), pl.ds(col0, dh)])


def kernel(x, edge_index, edge_weight, embedding, W, b):
    n_nodes, d_in = embedding.shape
    d_out = W.shape[1]
    n_edges = edge_weight.shape[0]
    bsz = x.shape[0]
    dh = d_out // NC

    h, h4 = pl.pallas_call(
        _matmul_body,
        grid=(10,),
        in_specs=[
            pl.BlockSpec((n_nodes // 10, d_in), lambda i: (i, 0)),
            pl.BlockSpec((d_in, d_out), lambda i: (0, 0)),
            pl.BlockSpec((1, d_out), lambda i: (0, 0)),
        ],
        out_specs=[
            pl.BlockSpec((n_nodes // 10, d_out), lambda i: (i, 0)),
            pl.BlockSpec((n_nodes // 10, d_out), lambda i: (i, 0)),
        ],
        out_shape=[
            jax.ShapeDtypeStruct((n_nodes, d_out), jnp.float32),
            jax.ShapeDtypeStruct((n_nodes, d_out), jnp.float32),
        ],
    )(embedding, W, b.reshape(1, d_out))

    n_groups = n_edges // GW
    b_groups = bsz // GW
    src2 = edge_index[0].reshape(n_groups, GW)
    dst2 = edge_index[1].reshape(n_groups, GW)
    w2 = edge_weight.reshape(n_groups, GW)
    x2 = x.reshape(b_groups, GW)

    mesh = plsc.VectorSubcoreMesh(core_axis_name="c", subcore_axis_name="s")
    propagate = pl.kernel(
        functools.partial(_propagate_body, n_nodes, n_groups, b_groups, dh),
        out_type=jax.ShapeDtypeStruct((bsz, d_out), jnp.float32),
        mesh=mesh,
        compiler_params=pltpu.CompilerParams(
            use_tc_tiling_on_sc=False, needs_layout_passes=False),
        scratch_types=[
            pltpu.VMEM_SHARED((n_nodes, dh), jnp.float32),
            pltpu.VMEM_SHARED((n_nodes, dh), jnp.float32),
            pltpu.VMEM((1, GW), jnp.int32),
            pltpu.VMEM((1, GW), jnp.int32),
            pltpu.VMEM((1, GW), jnp.float32),
            pltpu.VMEM((GW, dh), jnp.float32),
            pltpu.VMEM((1, GW), jnp.int32),
            pltpu.VMEM((GW, dh), jnp.float32),
        ],
    )
    rows = propagate(h, h4, src2, dst2, w2, x2)

    out = pl.pallas_call(
        _normalize_body,
        grid=(16,),
        in_specs=[pl.BlockSpec((bsz // 16, d_out), lambda i: (i, 0))],
        out_specs=pl.BlockSpec((bsz // 16, d_out), lambda i: (i, 0)),
        out_shape=jax.ShapeDtypeStruct((bsz, d_out), jnp.float32),
    )(rows)
    return out


# speedup vs baseline: 6.9652x; 1.9063x over previous
import functools

import jax
import jax.numpy as jnp
from jax import lax
from jax.experimental import pallas as pl
from jax.experimental.pallas import tpu as pltpu
from jax.experimental.pallas import tpu_sc as plsc

LAMDA = 0.8
ALPHA = (1.0 - LAMDA) / LAMDA

NC = 2
NS = 16
GW = 128
GPS = 158
PREF = 3


def _matmul_body(emb_ref, w_ref, b_ref, h_ref, h4_ref):
    h = jax.lax.dot_general(
        emb_ref[...], w_ref[...], (((1,), (0,)), ((), ())),
        precision=jax.lax.Precision.HIGHEST,
        preferred_element_type=jnp.float32) + b_ref[...]
    h_ref[...] = h
    h4_ref[...] = ALPHA * h


def _normalize_body(r_ref, o_ref):
    r = r_ref[...]
    norm = jnp.sqrt(jnp.sum(r * r, axis=1, keepdims=True))
    o_ref[...] = r / jnp.maximum(norm, 1e-12)


def _propagate_body(n_nodes, b_groups, dh,
                    h_hbm, h4_hbm, pk_hbm, x_hbm, out_hbm,
                    hs, acc, ebuf, rows, xq, obuf, sem_e, sem_g, sem_s):
    cid = lax.axis_index("c")
    sid = lax.axis_index("s")
    col0 = cid * dh
    rows_per = n_nodes // NS
    gb = sid * GPS

    def wait_rows(sem):
        pltpu.make_async_copy(
            h_hbm.at[pl.ds(0, GW), pl.ds(0, dh)], rows.at[0], sem).wait()

    def wait_ebuf(sem):
        pltpu.make_async_copy(pk_hbm.at[0], ebuf.at[0], sem).wait()

    r0 = sid * rows_per
    pltpu.sync_copy(h_hbm.at[pl.ds(r0, rows_per), pl.ds(col0, dh)],
                    hs.at[pl.ds(r0, rows_per)])
    pltpu.sync_copy(h4_hbm.at[pl.ds(r0, rows_per), pl.ds(col0, dh)],
                    acc.at[pl.ds(r0, rows_per)])
    plsc.subcore_barrier()

    def stage(t, b2, b4, first=False):
        wait_rows(sem_g)
        wait_ebuf(sem_e)
        if not first:
            wait_rows(sem_s)
        pltpu.async_copy(hs.at[ebuf.at[(b4 + 1) % 4, 0]],
                         rows.at[(b2 + 1) % 2], sem_g)
        pltpu.async_copy(pk_hbm.at[gb + t + 3], ebuf.at[(b4 + 3) % 4], sem_e)

        @pl.loop(0, GW, step=4)
        def _(e0):
            two16 = jnp.full((16,), 2, jnp.int32)
            for u in range(4):
                e = e0 + u
                wv = plsc.bitcast(
                    plsc.load_gather(ebuf.at[b4],
                                     [two16, jnp.broadcast_to(e, (16,))]),
                    jnp.float32)
                for j in range(dh // 16):
                    sl = pl.ds(j * 16, 16)
                    rows[b2, e, sl] = rows[b2, e, sl] * wv

        pltpu.async_copy(rows.at[b2], acc.at[ebuf.at[b4, 1]], sem_s, add=True)

    pltpu.async_copy(pk_hbm.at[gb], ebuf.at[0], sem_e)
    pltpu.async_copy(pk_hbm.at[gb + 1], ebuf.at[1], sem_e)
    pltpu.async_copy(pk_hbm.at[gb + 2], ebuf.at[2], sem_e)
    wait_ebuf(sem_e)
    pltpu.async_copy(hs.at[ebuf.at[0, 0]], rows.at[0], sem_g)

    stage(0, 0, 0, first=True)

    @pl.loop(1, GPS - 1, step=4)
    def _(t0):
        for b in range(4):
            stage(t0 + b, (1 + b) % 2, (1 + b) % 4)

    stage(GPS - 1, (GPS - 1) % 2, (GPS - 1) % 4)

    wait_rows(sem_s)
    wait_rows(sem_g)
    wait_ebuf(sem_e)
    wait_ebuf(sem_e)
    plsc.subcore_barrier()

    @pl.loop(sid, b_groups, step=NS)
    def _(g):
        pltpu.sync_copy(x_hbm.at[pl.ds(g, 1)], xq)
        pltpu.sync_copy(acc.at[xq.at[0]], obuf)
        pltpu.sync_copy(obuf, out_hbm.at[pl.ds(g * GW, GW), pl.ds(col0, dh)])


def kernel(x, edge_index, edge_weight, embedding, W, b):
    n_nodes, d_in = embedding.shape
    d_out = W.shape[1]
    n_edges = edge_weight.shape[0]
    bsz = x.shape[0]
    dh = d_out // NC

    h, h4 = pl.pallas_call(
        _matmul_body,
        grid=(10,),
        in_specs=[
            pl.BlockSpec((n_nodes // 10, d_in), lambda i: (i, 0)),
            pl.BlockSpec((d_in, d_out), lambda i: (0, 0)),
            pl.BlockSpec((1, d_out), lambda i: (0, 0)),
        ],
        out_specs=[
            pl.BlockSpec((n_nodes // 10, d_out), lambda i: (i, 0)),
            pl.BlockSpec((n_nodes // 10, d_out), lambda i: (i, 0)),
        ],
        out_shape=[
            jax.ShapeDtypeStruct((n_nodes, d_out), jnp.float32),
            jax.ShapeDtypeStruct((n_nodes, d_out), jnp.float32),
        ],
    )(embedding, W, b.reshape(1, d_out))

    n_groups = NS * GPS + PREF
    pad = n_groups * GW - n_edges
    src_p = jnp.concatenate([edge_index[0], jnp.zeros((pad,), jnp.int32)])
    dst_p = jnp.concatenate([edge_index[1], jnp.zeros((pad,), jnp.int32)])
    w_p = jnp.concatenate([edge_weight, jnp.zeros((pad,), jnp.float32)])
    pk = jnp.stack([src_p.reshape(n_groups, GW),
                    dst_p.reshape(n_groups, GW),
                    lax.bitcast_convert_type(w_p, jnp.int32)
                       .reshape(n_groups, GW)], axis=1)

    b_groups = bsz // GW
    x2 = x.reshape(b_groups, GW)

    mesh = plsc.VectorSubcoreMesh(core_axis_name="c", subcore_axis_name="s")
    propagate = pl.kernel(
        functools.partial(_propagate_body, n_nodes, b_groups, dh),
        out_type=jax.ShapeDtypeStruct((bsz, d_out), jnp.float32),
        mesh=mesh,
        compiler_params=pltpu.CompilerParams(
            use_tc_tiling_on_sc=False, needs_layout_passes=False),
        scratch_types=[
            pltpu.VMEM_SHARED((n_nodes, dh), jnp.float32),
            pltpu.VMEM_SHARED((n_nodes, dh), jnp.float32),
            pltpu.VMEM((4, 3, GW), jnp.int32),
            pltpu.VMEM((2, GW, dh), jnp.float32),
            pltpu.VMEM((1, GW), jnp.int32),
            pltpu.VMEM((GW, dh), jnp.float32),
            pltpu.SemaphoreType.DMA,
            pltpu.SemaphoreType.DMA,
            pltpu.SemaphoreType.DMA,
        ],
    )
    rows = propagate(h, h4, pk, x2)

    out = pl.pallas_call(
        _normalize_body,
        grid=(16,),
        in_specs=[pl.BlockSpec((bsz // 16, d_out), lambda i: (i, 0))],
        out_specs=pl.BlockSpec((bsz // 16, d_out), lambda i: (i, 0)),
        out_shape=jax.ShapeDtypeStruct((bsz, d_out), jnp.float32),
    )(rows)
    return out
